# baseline (speedup 1.0000x reference)
import jax
import jax.numpy as jnp
from jax import lax
from jax.experimental import pallas as pl
from jax.experimental.pallas import tpu as pltpu
from jax.experimental.pallas import tpu_sc as plsc

ND = 2000
NPRO = 8000
NV = ND + NPRO
FEAT = 128
E = 320000
NTILES = 32
EPT = E // NTILES
CH = 128
BUF = EPT + 2 * CH
NACC = 1
NC = 2048
AROWS = ND + 8


def _mlp_table_body(dF_ref, W1_ref, b1_ref, W2_ref, b2_ref, pE_ref, out_ref):
    h = jnp.maximum(dF_ref[...] @ W1_ref[...] + b1_ref[...][None, :], 0.0)
    h = jnp.maximum(h @ W2_ref[...] + b2_ref[...][None, :], 0.0)
    out_ref[0:ND, :] = h
    out_ref[ND:NV, :] = pE_ref[...]


def _sc_segsum_body(edge_ref, table_ref, zeros_ref, zc_ref,
                    sums_ref, cnts_ref,
                    src_in, dst_in, src_c, dst_c, didx, rows, cnt_l,
                    acc_sh, sg):
    cid = lax.axis_index("c")
    sid = lax.axis_index("s")
    wid = sid * 2 + cid

    with jax.named_scope("sc_init"):
        @pl.when(sid == 0)
        def _():
            pltpu.sync_copy(zeros_ref, acc_sh)

        pltpu.sync_copy(zc_ref, cnt_l)

        plsc.subcore_barrier()

        pltpu.sync_copy(edge_ref.at[0, wid], src_in)
        pltpu.sync_copy(edge_ref.at[1, wid], dst_in)

    ones16 = jnp.ones((16,), jnp.float32)

    def filt(i, off):
        sv = src_in[pl.ds(i * 16, 16)]
        dv = dst_in[pl.ds(i * 16, 16)]
        m = dv < ND
        plsc.store_compressed(src_c.at[pl.ds(off, 16)], sv, mask=m)
        plsc.store_compressed(dst_c.at[pl.ds(off, 16)], dv, mask=m)
        dvc = jnp.minimum(dv, NC - 1)
        plsc.addupdate_scatter(cnt_l, [dvc], ones16, mask=m)
        return off + jnp.sum(m.astype(jnp.int32))

    with jax.named_scope("sc_filter"):
        cnt = lax.fori_loop(0, EPT // 16, filt, 0)

    zsrc = jnp.zeros((16,), jnp.int32)
    tdst = jnp.full((16,), ND, jnp.int32)
    for k in range(CH // 16):
        src_c[pl.ds(cnt + k * 16, 16)] = zsrc
        dst_c[pl.ds(cnt + k * 16, 16)] = tdst

    nch = lax.shift_right_logical(cnt + (CH - 1), 7)
    arow0 = (sid % NACC) * AROWS

    def chunk(j, carry):
        dma = pltpu.async_copy(
            table_ref.at[src_c.at[pl.ds(j * CH, CH)]], rows, sg)
        for v in range(CH // 16):
            didx[pl.ds(v * 16, 16)] = (
                dst_c[pl.ds(j * CH + v * 16, 16)] + arow0)
        dma.wait()
        pltpu.sync_copy(rows, acc_sh.at[didx], add=True)
        return carry

    with jax.named_scope("sc_chunks"):
        lax.fori_loop(0, nch, chunk, 0)

    pltpu.sync_copy(cnt_l, cnts_ref.at[wid])

    plsc.subcore_barrier()

    @pl.when(sid == 0)
    def _():
        for a in range(NACC):
            pltpu.sync_copy(acc_sh.at[pl.ds(a * AROWS, ND)],
                            sums_ref.at[cid, a])


def _final_body(sums_ref, cnt_ref, dF_ref, Wl_ref, bl_ref, Wr_ref, out_ref):
    s = jnp.sum(sums_ref[...], axis=(0, 1))
    cnt = jnp.sum(cnt_ref[...], axis=0)[0:ND]
    mean = s / jnp.maximum(cnt, 1.0)[:, None]
    out_ref[...] = jnp.maximum(
        mean @ Wl_ref[...] + bl_ref[...][None, :] + dF_ref[...] @ Wr_ref[...],
        0.0)


def kernel(edge_index, drugFeatures, W1, b1, W2, b2, protEmb,
           Wl0, bl0, Wr0, Wl1, bl1, Wr1):
    ei = edge_index.astype(jnp.int32).reshape(2, NTILES, EPT)

    table = pl.pallas_call(
        _mlp_table_body,
        out_shape=jax.ShapeDtypeStruct((NV, FEAT), jnp.float32),
    )(drugFeatures, W1, b1, W2, b2, protEmb)

    zeros = jnp.zeros((NACC * AROWS, FEAT), jnp.float32)
    zc = jnp.zeros((NC,), jnp.float32)
    mesh = plsc.VectorSubcoreMesh(core_axis_name="c", subcore_axis_name="s")
    sums, cnts = pl.kernel(
        _sc_segsum_body,
        out_type=(
            jax.ShapeDtypeStruct((2, NACC, ND, FEAT), jnp.float32),
            jax.ShapeDtypeStruct((NTILES, NC), jnp.float32),
        ),
        mesh=mesh,
        compiler_params=pltpu.CompilerParams(needs_layout_passes=False),
        scratch_types=[
            pltpu.VMEM((EPT,), jnp.int32),
            pltpu.VMEM((EPT,), jnp.int32),
            pltpu.VMEM((BUF,), jnp.int32),
            pltpu.VMEM((BUF,), jnp.int32),
            pltpu.VMEM((CH,), jnp.int32),
            pltpu.VMEM((CH, FEAT), jnp.float32),
            pltpu.VMEM((NC,), jnp.float32),
            pltpu.VMEM_SHARED((NACC * AROWS, FEAT), jnp.float32),
            pltpu.SemaphoreType.DMA,
        ],
    )(ei, table, zeros, zc)

    out = pl.pallas_call(
        _final_body,
        out_shape=jax.ShapeDtypeStruct((ND, FEAT), jnp.float32),
        grid=(1,),
        in_specs=[
            pl.BlockSpec((2, NACC, ND, FEAT), lambda i: (0, 0, 0, 0)),
            pl.BlockSpec((NTILES, NC), lambda i: (0, 0)),
            pl.BlockSpec((ND, FEAT), lambda i: (0, 0)),
            pl.BlockSpec((FEAT, FEAT), lambda i: (0, 0)),
            pl.BlockSpec((FEAT,), lambda i: (0,)),
            pl.BlockSpec((FEAT, FEAT), lambda i: (0, 0)),
        ],
        out_specs=pl.BlockSpec((ND, FEAT), lambda i: (0, 0)),
    )(sums, cnts, table, Wl1, bl1, Wr1)
    return out
